# baseline (speedup 1.0000x reference)
import functools

import jax
import jax.numpy as jnp
from jax import lax
from jax.experimental import pallas as pl
from jax.experimental.pallas import tpu as pltpu
from jax.experimental.pallas import tpu_sc as plsc

R = 64
RRR = R * R * R
N = 20000
C = 128
W = C // 2
G = 16
NGROUPS = N // G
NW = 32
GPW = (NGROUPS + NW - 1) // NW
CAP = 27 * G
NBROWS = 384
ZSLOT = NBROWS - 1
MAXOFF = NBROWS - 32

OFFSETS = [(0, 0, 0), (0, 0, 1), (0, 1, 0), (0, 1, 1),
           (1, 0, 0), (1, 0, 1), (1, 1, 0), (1, 1, 1)]
M_OFF = [(ex, ey, ez) for ex in (-1, 0, 1) for ey in (-1, 0, 1) for ez in (-1, 0, 1)]
_M_IDX = {m: i for i, m in enumerate(M_OFF)}


def _plan():
    plan = []
    for o in OFFSETS:
        terms = []
        for d in OFFSETS:
            e = tuple((d[a] - 1 if o[a] == 0 else d[a]) for a in range(3))
            w = 1.0
            for a in range(3):
                near = (o[a] == 0 and d[a] == 1) or (o[a] == 1 and d[a] == 0)
                w *= 0.75 if near else 0.25
            terms.append((_M_IDX[e], w))
        plan.append(terms)
    return plan


PLAN = _plan()
PLAN_BY_M = [[] for _ in range(27)]
for _oi, _terms in enumerate(PLAN):
    for _mi, _wt in _terms:
        PLAN_BY_M[_mi].append((_oi, _wt))

_mesh = plsc.VectorSubcoreMesh(core_axis_name="c", subcore_axis_name="s")
_params = pltpu.CompilerParams(needs_layout_passes=False)


@functools.partial(
    pl.kernel,
    out_type=(jax.ShapeDtypeStruct((RRR + G,), jnp.int32),
              jax.ShapeDtypeStruct((N,), jnp.int32)),
    mesh=_mesh,
    compiler_params=_params,
    scratch_types=[
        pltpu.VMEM((G,), jnp.int32),
        pltpu.VMEM((G,), jnp.int32),
        pltpu.VMEM((G,), jnp.int32),
        pltpu.VMEM((G,), jnp.int32),
        pltpu.VMEM((G,), jnp.int32),
        pltpu.SemaphoreType.DMA,
    ],
)
def _build(ci, cj, ck, vol, flat_of, vi, vj, vk, fv, av, sem):
    wid = lax.axis_index("s") * 2 + lax.axis_index("c")

    def body(t, carry):
        g = wid + t * NW

        @pl.when(g < NGROUPS)
        def _():
            pltpu.sync_copy(ci.at[pl.ds(g * G, G)], vi)
            pltpu.sync_copy(cj.at[pl.ds(g * G, G)], vj)
            pltpu.sync_copy(ck.at[pl.ds(g * G, G)], vk)
            fv[...] = (vi[...] * R + vj[...]) * R + vk[...]
            av[...] = lax.iota(jnp.int32, G) + g * G
            pltpu.sync_copy(fv, flat_of.at[pl.ds(g * G, G)])
            pltpu.async_copy(av, vol.at[fv], sem).wait()

        return carry

    lax.fori_loop(0, GPW, body, 0)


@functools.partial(
    pl.kernel,
    out_type=jax.ShapeDtypeStruct((8 * N, C), jnp.float32),
    mesh=_mesh,
    compiler_params=_params,
    scratch_types=[
        pltpu.VMEM((N,), jnp.int32),
        pltpu.VMEM((G,), jnp.int32),
        pltpu.VMEM((G,), jnp.int32),
        pltpu.VMEM((G,), jnp.int32),
        pltpu.VMEM((CAP,), jnp.int32),
        pltpu.VMEM((CAP,), jnp.int32),
        pltpu.VMEM((CAP + G,), jnp.int32),
        pltpu.VMEM((CAP,), jnp.int32),
        pltpu.VMEM((2, G, C), jnp.float32),
        pltpu.VMEM((NBROWS * (C + 1),), jnp.float32),
        pltpu.VMEM((G * (8 * C + 17),), jnp.float32),
        pltpu.VMEM((8 * G, C), jnp.float32),
        pltpu.SemaphoreType.DMA,
        pltpu.SemaphoreType.DMA,
    ],
)
def _main(ci, cj, ck, vol, flat_of, featw, out,
          fo_v, vi, vj, vk, nfb, rb, rcl, posb, stage, nbrc, obo, ob, sem,
          sem2):
    wid = lax.axis_index("s") * 2 + lax.axis_index("c")
    pltpu.sync_copy(flat_of, fo_v)
    zf = jnp.zeros((16,), jnp.float32)
    for k in range(C // 16):
        nbrc[pl.ds(ZSLOT * (C + 1) + 16 * k, 16)] = zf
    zi = jnp.zeros((16,), jnp.int32)
    for k in range((CAP + G) // 16):
        rcl[pl.ds(16 * k, 16)] = zi

    def body(t, carry):
        g = wid + t * NW

        @pl.when(g < NGROUPS)
        def _():
            pltpu.async_copy(ci.at[pl.ds(g * G, G)], vi, sem)
            pltpu.async_copy(cj.at[pl.ds(g * G, G)], vj, sem)
            pltpu.async_copy(ck.at[pl.ds(g * G, G)], vk, sem)
            pltpu.make_async_copy(ci.at[pl.ds(g * G, G)], vi, sem).wait()
            pltpu.make_async_copy(cj.at[pl.ds(g * G, G)], vj, sem).wait()
            pltpu.make_async_copy(ck.at[pl.ds(g * G, G)], vk, sem).wait()
            i = vi[...]
            j = vj[...]
            k_ = vk[...]
            for m, (ex, ey, ez) in enumerate(M_OFF):
                ni = i + ex
                nj = j + ey
                nk = k_ + ez
                inb = ((ni >= 0) & (ni < R) & (nj >= 0) & (nj < R)
                       & (nk >= 0) & (nk < R))
                nf = jnp.where(inb, (ni * R + nj) * R + nk, RRR)
                nfb[pl.ds(m * G, G)] = nf
            pltpu.async_copy(vol.at[nfb], rb, sem)
            pltpu.sync_copy(featw.at[pl.ds(g * G, G)], stage.at[0])
            for v in range(G):
                for k in range(C // 16):
                    nbrc[pl.ds(v * (C + 1) + 16 * k, 16)] = (
                        stage[0, v, pl.ds(16 * k, 16)])
            iot = lax.iota(jnp.int32, G)
            posb[pl.ds(13 * G, G)] = iot
            pltpu.make_async_copy(vol.at[nfb], rb, sem).wait()
            off = jnp.int32(G)
            for m in range(27):
                if m == 13:
                    continue
                r = rb[pl.ds(m * G, G)]
                rc = jnp.clip(r, 0, N - 1)
                fo = plsc.load_gather(fo_v, [rc])
                valid = (r >= 0) & (r < N) & (fo == nfb[pl.ds(m * G, G)])
                v01 = jnp.where(valid, 1, 0).astype(jnp.int32)
                incl = plsc.cumsum(v01)
                pos = jnp.where(valid, off + incl - v01, ZSLOT)
                posb[pl.ds(m * G, G)] = jnp.minimum(pos, ZSLOT)
                plsc.store_compressed(rcl.at[pl.ds(off, G)], rc, mask=valid)
                off = jnp.minimum(off + jnp.sum(v01), MAXOFF)
            nch = (off - G + (G - 1)) >> 4

            @pl.when(nch > 0)
            def _fire0():
                pltpu.async_copy(featw.at[rcl[pl.ds(G, G)]], stage.at[0], sem)

            def gbody(tt, gcarry):
                par = tt & 1
                pltpu.make_async_copy(
                    featw.at[rcl[pl.ds(G + tt * G, G)]],
                    stage.at[par], sem).wait()

                @pl.when(tt + 1 < nch)
                def _firenext():
                    pltpu.async_copy(
                        featw.at[rcl[pl.ds(G + (tt + 1) * G, G)]],
                        stage.at[1 - par], sem)

                base = (G + tt * G) * (C + 1)
                for v in range(G):
                    for k in range(C // 16):
                        nbrc[pl.ds(base + v * (C + 1) + 16 * k, 16)] = (
                            stage[par, v, pl.ds(16 * k, 16)])
                return gcarry

            lax.fori_loop(0, nch, gbody, 0, unroll=False)
            pidx = [posb[pl.ds(m * G, G)] * (C + 1) for m in range(27)]
            obase = lax.iota(jnp.int32, G) * (8 * C + 17)

            @plsc.parallel_loop(0, C, step=1, unroll=4)
            def wbody(w):
                wvec = jnp.full((G,), w, jnp.int32)
                acc = [None] * 8
                for m in range(27):
                    val = plsc.load_gather(nbrc, [pidx[m] + wvec])
                    for (oi, wt) in PLAN_BY_M[m]:
                        tv = val * jnp.float32(wt)
                        acc[oi] = tv if acc[oi] is None else acc[oi] + tv
                for oi in range(8):
                    plsc.store_scatter(obo, [obase + (oi * C) + wvec], acc[oi])

            @pl.when(t > 0)
            def _drain():
                pltpu.make_async_copy(
                    ob, out.at[pl.ds((g - NW) * 8 * G, 8 * G)], sem2).wait()

            def rbody(v, rcarry):
                for k in range(8 * C // 16):
                    ob[v * 8 + (k // 8), pl.ds((k % 8) * 16, 16)] = (
                        obo[pl.ds(v * (8 * C + 17) + 16 * k, 16)])
                return rcarry

            lax.fori_loop(0, G, rbody, 0)
            pltpu.async_copy(ob, out.at[pl.ds(g * 8 * G, 8 * G)], sem2)

        return carry

    lax.fori_loop(0, GPW, body, 0)
    gl = wid + ((NGROUPS - 1 - wid) // NW) * NW
    pltpu.make_async_copy(ob, out.at[pl.ds(gl * 8 * G, 8 * G)], sem2).wait()


def kernel(feat, coords):
    coords = coords.astype(jnp.int32)
    ci = coords[:, 0]
    cj = coords[:, 1]
    ck = coords[:, 2]
    vol, flat_of = _build(ci, cj, ck)
    return _main(ci, cj, ck, vol, flat_of, feat)


# speedup vs baseline: 1.2393x; 1.2393x over previous
import functools

import jax
import jax.numpy as jnp
from jax import lax
from jax.experimental import pallas as pl
from jax.experimental.pallas import tpu as pltpu
from jax.experimental.pallas import tpu_sc as plsc

R = 64
RRR = R * R * R
N = 20000
C = 128
W = C // 2
G = 16
NGROUPS = N // G
NW = 32
GPW = (NGROUPS + NW - 1) // NW
CAP = 27 * G
NBROWS = 384
ZSLOT = NBROWS - 1
MAXOFF = NBROWS - 32

OFFSETS = [(0, 0, 0), (0, 0, 1), (0, 1, 0), (0, 1, 1),
           (1, 0, 0), (1, 0, 1), (1, 1, 0), (1, 1, 1)]
M_OFF = [(ex, ey, ez) for ex in (-1, 0, 1) for ey in (-1, 0, 1) for ez in (-1, 0, 1)]
_M_IDX = {m: i for i, m in enumerate(M_OFF)}


def _plan():
    plan = []
    for o in OFFSETS:
        terms = []
        for d in OFFSETS:
            e = tuple((d[a] - 1 if o[a] == 0 else d[a]) for a in range(3))
            w = 1.0
            for a in range(3):
                near = (o[a] == 0 and d[a] == 1) or (o[a] == 1 and d[a] == 0)
                w *= 0.75 if near else 0.25
            terms.append((_M_IDX[e], w))
        plan.append(terms)
    return plan


PLAN = _plan()
PLAN_BY_M = [[] for _ in range(27)]
for _oi, _terms in enumerate(PLAN):
    for _mi, _wt in _terms:
        PLAN_BY_M[_mi].append((_oi, _wt))

_mesh = plsc.VectorSubcoreMesh(core_axis_name="c", subcore_axis_name="s")
_params = pltpu.CompilerParams(needs_layout_passes=False)


@functools.partial(
    pl.kernel,
    out_type=(jax.ShapeDtypeStruct((RRR + G,), jnp.int32),
              jax.ShapeDtypeStruct((N,), jnp.int32)),
    mesh=_mesh,
    compiler_params=_params,
    scratch_types=[
        pltpu.VMEM((G,), jnp.int32),
        pltpu.VMEM((G,), jnp.int32),
        pltpu.VMEM((G,), jnp.int32),
        pltpu.VMEM((G,), jnp.int32),
        pltpu.VMEM((G,), jnp.int32),
        pltpu.SemaphoreType.DMA,
    ],
)
def _build(ci, cj, ck, vol, flat_of, vi, vj, vk, fv, av, sem):
    wid = lax.axis_index("s") * 2 + lax.axis_index("c")

    def body(t, carry):
        g = wid + t * NW

        @pl.when(g < NGROUPS)
        def _():
            pltpu.sync_copy(ci.at[pl.ds(g * G, G)], vi)
            pltpu.sync_copy(cj.at[pl.ds(g * G, G)], vj)
            pltpu.sync_copy(ck.at[pl.ds(g * G, G)], vk)
            fv[...] = (vi[...] * R + vj[...]) * R + vk[...]
            av[...] = lax.iota(jnp.int32, G) + g * G
            pltpu.sync_copy(fv, flat_of.at[pl.ds(g * G, G)])
            pltpu.async_copy(av, vol.at[fv], sem).wait()

        return carry

    lax.fori_loop(0, GPW, body, 0)


@functools.partial(
    pl.kernel,
    out_type=jax.ShapeDtypeStruct((8 * N, C), jnp.float32),
    mesh=_mesh,
    compiler_params=_params,
    scratch_types=[
        pltpu.VMEM((N,), jnp.int32),
        pltpu.VMEM((G,), jnp.int32),
        pltpu.VMEM((G,), jnp.int32),
        pltpu.VMEM((G,), jnp.int32),
        pltpu.VMEM((CAP,), jnp.int32),
        pltpu.VMEM((CAP,), jnp.int32),
        pltpu.VMEM((CAP + G,), jnp.int32),
        pltpu.VMEM((CAP,), jnp.int32),
        pltpu.VMEM((2, G, C), jnp.float32),
        pltpu.VMEM((NBROWS * (C + 1),), jnp.float32),
        pltpu.VMEM((G * (8 * C + 17),), jnp.float32),
        pltpu.VMEM((8 * G, C), jnp.float32),
        pltpu.SemaphoreType.DMA,
        pltpu.SemaphoreType.DMA,
    ],
)
def _main(ci, cj, ck, vol, flat_of, featw, out,
          fo_v, vi, vj, vk, nfb, rb, rcl, posb, stage, nbrc, obo, ob, sem,
          sem2):
    wid = lax.axis_index("s") * 2 + lax.axis_index("c")
    pltpu.sync_copy(flat_of, fo_v)
    zf = jnp.zeros((16,), jnp.float32)
    for k in range(C // 16):
        nbrc[pl.ds(ZSLOT * (C + 1) + 16 * k, 16)] = zf
    zi = jnp.zeros((16,), jnp.int32)
    for k in range((CAP + G) // 16):
        rcl[pl.ds(16 * k, 16)] = zi

    def body(t, carry):
        g = wid + t * NW

        @pl.when(g < NGROUPS)
        def _():
            pltpu.async_copy(ci.at[pl.ds(g * G, G)], vi, sem)
            pltpu.async_copy(cj.at[pl.ds(g * G, G)], vj, sem)
            pltpu.async_copy(ck.at[pl.ds(g * G, G)], vk, sem)
            pltpu.make_async_copy(ci.at[pl.ds(g * G, G)], vi, sem).wait()
            pltpu.make_async_copy(cj.at[pl.ds(g * G, G)], vj, sem).wait()
            pltpu.make_async_copy(ck.at[pl.ds(g * G, G)], vk, sem).wait()
            i = vi[...]
            j = vj[...]
            k_ = vk[...]
            for m, (ex, ey, ez) in enumerate(M_OFF):
                ni = i + ex
                nj = j + ey
                nk = k_ + ez
                inb = ((ni >= 0) & (ni < R) & (nj >= 0) & (nj < R)
                       & (nk >= 0) & (nk < R))
                nf = jnp.where(inb, (ni * R + nj) * R + nk, RRR)
                nfb[pl.ds(m * G, G)] = nf
            pltpu.async_copy(vol.at[nfb], rb, sem)
            pltpu.sync_copy(featw.at[pl.ds(g * G, G)], stage.at[0])
            for v in range(G):
                for k in range(C // 16):
                    nbrc[pl.ds(v * (C + 1) + 16 * k, 16)] = (
                        stage[0, v, pl.ds(16 * k, 16)])
            iot = lax.iota(jnp.int32, G)
            posb[pl.ds(13 * G, G)] = iot
            pltpu.make_async_copy(vol.at[nfb], rb, sem).wait()
            off = jnp.int32(G)
            for m in range(27):
                if m == 13:
                    continue
                r = rb[pl.ds(m * G, G)]
                rc = jnp.clip(r, 0, N - 1)
                fo = plsc.load_gather(fo_v, [rc])
                valid = (r >= 0) & (r < N) & (fo == nfb[pl.ds(m * G, G)])
                v01 = jnp.where(valid, 1, 0).astype(jnp.int32)
                incl = plsc.cumsum(v01)
                pos = jnp.where(valid, off + incl - v01, ZSLOT)
                posb[pl.ds(m * G, G)] = jnp.minimum(pos, ZSLOT)
                plsc.store_compressed(rcl.at[pl.ds(off, G)], rc, mask=valid)
                off = jnp.minimum(off + jnp.sum(v01), MAXOFF)
            nch = (off - G + (G - 1)) >> 4

            @pl.when(nch > 0)
            def _fire0():
                pltpu.async_copy(featw.at[rcl[pl.ds(G, G)]], stage.at[0], sem)

            def gbody(tt, gcarry):
                par = tt & 1
                pltpu.make_async_copy(
                    featw.at[rcl[pl.ds(G + tt * G, G)]],
                    stage.at[par], sem).wait()

                @pl.when(tt + 1 < nch)
                def _firenext():
                    pltpu.async_copy(
                        featw.at[rcl[pl.ds(G + (tt + 1) * G, G)]],
                        stage.at[1 - par], sem)

                base = (G + tt * G) * (C + 1)
                for v in range(G):
                    for k in range(C // 16):
                        nbrc[pl.ds(base + v * (C + 1) + 16 * k, 16)] = (
                            stage[par, v, pl.ds(16 * k, 16)])
                return gcarry

            lax.fori_loop(0, nch, gbody, 0, unroll=False)
            pidx = [posb[pl.ds(m * G, G)] * (C + 1) for m in range(27)]
            obase = lax.iota(jnp.int32, G) * (8 * C + 17)

            @plsc.parallel_loop(0, C, step=1, unroll=2)
            def wbody(w):
                wvec = jnp.full((G,), w, jnp.int32)
                acc = [None] * 8
                for m in range(27):
                    val = plsc.load_gather(nbrc, [pidx[m] + wvec])
                    for (oi, wt) in PLAN_BY_M[m]:
                        tv = val * jnp.float32(wt)
                        acc[oi] = tv if acc[oi] is None else acc[oi] + tv
                for oi in range(8):
                    plsc.store_scatter(obo, [obase + (oi * C) + wvec], acc[oi])

            @pl.when(t > 0)
            def _drain():
                pltpu.make_async_copy(
                    ob, out.at[pl.ds((g - NW) * 8 * G, 8 * G)], sem2).wait()

            def rbody(v, rcarry):
                for k in range(8 * C // 16):
                    ob[v * 8 + (k // 8), pl.ds((k % 8) * 16, 16)] = (
                        obo[pl.ds(v * (8 * C + 17) + 16 * k, 16)])
                return rcarry

            lax.fori_loop(0, G, rbody, 0)
            pltpu.async_copy(ob, out.at[pl.ds(g * 8 * G, 8 * G)], sem2)

        return carry

    lax.fori_loop(0, GPW, body, 0)
    gl = wid + ((NGROUPS - 1 - wid) // NW) * NW
    pltpu.make_async_copy(ob, out.at[pl.ds(gl * 8 * G, 8 * G)], sem2).wait()


def kernel(feat, coords):
    coords = coords.astype(jnp.int32)
    ci = coords[:, 0]
    cj = coords[:, 1]
    ck = coords[:, 2]
    vol, flat_of = _build(ci, cj, ck)
    return _main(ci, cj, ck, vol, flat_of, feat)
